# baseline (speedup 1.0000x reference)
import dataclasses

import jax
import jax.numpy as jnp
from jax import lax
from jax.experimental import pallas as pl
from jax.experimental.pallas import tpu as pltpu
from jax.experimental.pallas import tpu_sc as plsc

VOCAB = 13
L = 200
B = 16384
E = 32
H = 16
O = 4

NUM_TILES = 32
ROWS_PER_TILE = B // NUM_TILES
CHUNK_ROWS = 64
NUM_CHUNKS = ROWS_PER_TILE // CHUNK_ROWS
CHUNK_BINS = CHUNK_ROWS * 16
FULL_VECS = L // 16
TAIL_OFF = L - 16
TAIL_MASKED = 16 - (L - 16 * FULL_VECS)

STRIDE = 17
ROW_SPAN = 16 * STRIDE


def _sc_hist_kernel(x_hbm, cnt_hbm,
                    xv0, xv1, scr, out0, out1,
                    s_in0, s_in1, s_out0, s_out1):
    wid = lax.axis_index("s") * 2 + lax.axis_index("c")
    row_base = wid * ROWS_PER_TILE

    ones = jnp.full((16,), 1.0, dtype=jnp.float32)
    zeros = jnp.zeros((16,), dtype=jnp.float32)
    tail_mask = jnp.arange(16, dtype=jnp.int32) >= TAIL_MASKED
    laneoff = jnp.arange(16, dtype=jnp.int32) * STRIDE

    xbufs = [xv0, xv1]
    obufs = [out0, out1]
    in_sems = [s_in0, s_in1]
    out_sems = [s_out0, s_out1]

    in_copies = [None, None]
    out_copies = [None, None]

    @plsc.parallel_loop(0, CHUNK_ROWS * ROW_SPAN, step=16, unroll=8)
    def _zero(j):
        scr[pl.ds(j, 16)] = zeros

    in_copies[0] = pltpu.async_copy(
        x_hbm.at[pl.ds(row_base, CHUNK_ROWS)], xbufs[0], in_sems[0])

    for ch in range(NUM_CHUNKS):
        p = ch % 2
        if ch + 1 < NUM_CHUNKS:
            in_copies[1 - p] = pltpu.async_copy(
                x_hbm.at[pl.ds(row_base + (ch + 1) * CHUNK_ROWS, CHUNK_ROWS)],
                xbufs[1 - p], in_sems[1 - p])
        if out_copies[p] is not None:
            out_copies[p].wait()
        in_copies[p].wait()
        xv = xbufs[p]
        outv = obufs[p]

        @plsc.parallel_loop(0, CHUNK_ROWS, step=1, unroll=2)
        def _row(r, xv=xv, outv=outv):
            roff = laneoff + r * ROW_SPAN
            for k in range(FULL_VECS):
                vals = xv[r, pl.ds(k * 16, 16)]
                plsc.addupdate_scatter(scr, [vals + roff], ones)
            tail = xv[r, pl.ds(TAIL_OFF, 16)]
            plsc.addupdate_scatter(scr, [tail + roff], ones, mask=tail_mask)
            base = r * ROW_SPAN
            acc = scr[pl.ds(base, 16)]
            scr[pl.ds(base, 16)] = zeros
            for l in range(1, 16):
                acc = acc + scr[pl.ds(base + l * STRIDE, 16)]
                scr[pl.ds(base + l * STRIDE, 16)] = zeros
            outv[pl.ds(r * 16, 16)] = acc

        out_copies[p] = pltpu.async_copy(
            outv,
            cnt_hbm.at[pl.ds((row_base + ch * CHUNK_ROWS) * 16, CHUNK_BINS)],
            out_sems[p])

    for p in range(2):
        if out_copies[p] is not None:
            out_copies[p].wait()


def _sc_histogram(x):
    mesh = plsc.VectorSubcoreMesh(core_axis_name="c", subcore_axis_name="s")
    cp = pltpu.CompilerParams()
    if "needs_layout_passes" in pltpu.CompilerParams.__dataclass_fields__:
        cp = dataclasses.replace(cp, needs_layout_passes=False)
    f = pl.kernel(
        _sc_hist_kernel,
        out_type=jax.ShapeDtypeStruct((B * 16,), jnp.float32),
        mesh=mesh,
        scratch_types=[
            pltpu.VMEM((CHUNK_ROWS, L), jnp.int32),
            pltpu.VMEM((CHUNK_ROWS, L), jnp.int32),
            pltpu.VMEM((CHUNK_ROWS * ROW_SPAN,), jnp.float32),
            pltpu.VMEM((CHUNK_BINS,), jnp.float32),
            pltpu.VMEM((CHUNK_BINS,), jnp.float32),
            pltpu.SemaphoreType.DMA,
            pltpu.SemaphoreType.DMA,
            pltpu.SemaphoreType.DMA,
            pltpu.SemaphoreType.DMA,
        ],
        compiler_params=cp,
    )
    return f(x)


MLP_ROWS = 2048
MLP_BLK = MLP_ROWS * 16


def _mlp_kernel(cnt_ref, table_ref, w1_ref, b1_ref, w2_ref, b2_ref, out_ref):
    m = jnp.dot(table_ref[...], w1_ref[...].T,
                preferred_element_type=jnp.float32) * (1.0 / L)
    mp = jnp.concatenate([m, jnp.zeros((16 - VOCAB, H), jnp.float32)], axis=0)

    blk = jnp.equal(lax.broadcasted_iota(jnp.int32, (128, 128), 0) // 16,
                    lax.broadcasted_iota(jnp.int32, (128, 128), 1) // 16)
    m8 = jnp.tile(mp, (8, 8)) * blk.astype(jnp.float32)
    blk2 = jnp.equal(lax.broadcasted_iota(jnp.int32, (128, 128), 0) // 16,
                     lax.broadcasted_iota(jnp.int32, (128, 128), 1) // 16)
    w2p = jnp.concatenate(
        [w2_ref[...].T, jnp.zeros((H, 16 - O), jnp.float32)], axis=1)
    w28 = jnp.tile(w2p, (8, 8)) * blk2.astype(jnp.float32)
    b1t = jnp.tile(b1_ref[0], (8,))
    b2t = jnp.tile(jnp.concatenate(
        [b2_ref[0], jnp.zeros((16 - O,), jnp.float32)]), (8,))

    cnt = cnt_ref[...].reshape(MLP_BLK // 128, 128)
    h = jnp.maximum(
        jnp.dot(cnt, m8, preferred_element_type=jnp.float32) + b1t[None, :],
        0.0)
    out = jnp.dot(h, w28, preferred_element_type=jnp.float32) + b2t[None, :]
    out_ref[...] = out.reshape(MLP_BLK)


def _tc_mlp(counts_flat, table, W1, b1, W2, b2):
    out_pad = pl.pallas_call(
        _mlp_kernel,
        grid=(B // MLP_ROWS,),
        in_specs=[
            pl.BlockSpec((MLP_BLK,), lambda i: (i,)),
            pl.BlockSpec((VOCAB, E), lambda i: (0, 0)),
            pl.BlockSpec((H, E), lambda i: (0, 0)),
            pl.BlockSpec((1, H), lambda i: (0, 0)),
            pl.BlockSpec((O, H), lambda i: (0, 0)),
            pl.BlockSpec((1, O), lambda i: (0, 0)),
        ],
        out_specs=pl.BlockSpec((MLP_BLK,), lambda i: (i,)),
        out_shape=jax.ShapeDtypeStruct((B * 16,), jnp.float32),
        compiler_params=pltpu.CompilerParams(
            dimension_semantics=("arbitrary",),
        ),
    )(counts_flat, table, W1, b1.reshape(1, H), W2, b2.reshape(1, O))
    return out_pad.reshape(B, 16)[:, :O]


def kernel(x, table, W1, b1, W2, b2):
    counts_flat = _sc_histogram(x.astype(jnp.int32))
    return _tc_mlp(counts_flat, table, W1, b1, W2, b2)


# speedup vs baseline: 1.1326x; 1.1326x over previous
import dataclasses

import jax
import jax.numpy as jnp
from jax import lax
from jax.experimental import pallas as pl
from jax.experimental.pallas import tpu as pltpu
from jax.experimental.pallas import tpu_sc as plsc

VOCAB = 13
L = 200
B = 16384
E = 32
H = 16
O = 4

NUM_TILES = 32
ROWS_PER_TILE = B // NUM_TILES
CHUNK_ROWS = 128
NUM_CHUNKS = ROWS_PER_TILE // CHUNK_ROWS
CHUNK_BINS = CHUNK_ROWS * 16
FULL_VECS = L // 16
TAIL_OFF = L - 16
TAIL_MASKED = 16 - (L - 16 * FULL_VECS)


def _sc_hist_kernel(x_hbm, cnt_hbm,
                    xv0, xv1, cnt0, cnt1,
                    s_in0, s_in1, s_out0, s_out1):
    wid = lax.axis_index("s") * 2 + lax.axis_index("c")
    row_base = wid * ROWS_PER_TILE

    ones = jnp.full((16,), 1.0, dtype=jnp.float32)
    zeros = jnp.zeros((16,), dtype=jnp.float32)
    tail_mask = jnp.arange(16, dtype=jnp.int32) >= TAIL_MASKED

    xbufs = [xv0, xv1]
    cbufs = [cnt0, cnt1]
    in_sems = [s_in0, s_in1]
    out_sems = [s_out0, s_out1]

    in_copies = [None, None]
    out_copies = [None, None]

    in_copies[0] = pltpu.async_copy(
        x_hbm.at[pl.ds(row_base, CHUNK_ROWS)], xbufs[0], in_sems[0])

    for ch in range(NUM_CHUNKS):
        p = ch % 2
        if ch + 1 < NUM_CHUNKS:
            in_copies[1 - p] = pltpu.async_copy(
                x_hbm.at[pl.ds(row_base + (ch + 1) * CHUNK_ROWS, CHUNK_ROWS)],
                xbufs[1 - p], in_sems[1 - p])
        if out_copies[p] is not None:
            out_copies[p].wait()
        in_copies[p].wait()
        xv = xbufs[p]
        cntv = cbufs[p]

        @plsc.parallel_loop(0, CHUNK_BINS, step=16, unroll=8)
        def _zero(j, cntv=cntv):
            cntv[pl.ds(j, 16)] = zeros

        @plsc.parallel_loop(0, CHUNK_ROWS, step=1, unroll=2)
        def _row(r, xv=xv, cntv=cntv):
            roff = jnp.full((16,), r * 16, dtype=jnp.int32)
            for k in range(FULL_VECS):
                vals = xv[r, pl.ds(k * 16, 16)]
                plsc.addupdate_scatter(cntv, [vals + roff], ones)
            tail = xv[r, pl.ds(TAIL_OFF, 16)]
            plsc.addupdate_scatter(cntv, [tail + roff], ones, mask=tail_mask)

        out_copies[p] = pltpu.async_copy(
            cntv,
            cnt_hbm.at[pl.ds((row_base + ch * CHUNK_ROWS) * 16, CHUNK_BINS)],
            out_sems[p])

    for p in range(2):
        if out_copies[p] is not None:
            out_copies[p].wait()


def _sc_histogram(x):
    mesh = plsc.VectorSubcoreMesh(core_axis_name="c", subcore_axis_name="s")
    cp = pltpu.CompilerParams()
    if "needs_layout_passes" in pltpu.CompilerParams.__dataclass_fields__:
        cp = dataclasses.replace(cp, needs_layout_passes=False)
    f = pl.kernel(
        _sc_hist_kernel,
        out_type=jax.ShapeDtypeStruct((B * 16,), jnp.float32),
        mesh=mesh,
        scratch_types=[
            pltpu.VMEM((CHUNK_ROWS, L), jnp.int32),
            pltpu.VMEM((CHUNK_ROWS, L), jnp.int32),
            pltpu.VMEM((CHUNK_BINS,), jnp.float32),
            pltpu.VMEM((CHUNK_BINS,), jnp.float32),
            pltpu.SemaphoreType.DMA,
            pltpu.SemaphoreType.DMA,
            pltpu.SemaphoreType.DMA,
            pltpu.SemaphoreType.DMA,
        ],
        compiler_params=cp,
    )
    return f(x)


MLP_ROWS = 2048
MLP_BLK = MLP_ROWS * 16


def _mlp_kernel(cnt_ref, table_ref, w1_ref, b1_ref, w2_ref, b2_ref, out_ref):
    m = jnp.dot(table_ref[...], w1_ref[...].T,
                preferred_element_type=jnp.float32) * (1.0 / L)
    mp = jnp.concatenate([m, jnp.zeros((16 - VOCAB, H), jnp.float32)], axis=0)

    blk = jnp.equal(lax.broadcasted_iota(jnp.int32, (128, 128), 0) // 16,
                    lax.broadcasted_iota(jnp.int32, (128, 128), 1) // 16)
    blkf = blk.astype(jnp.float32)
    m8 = jnp.tile(mp, (8, 8)) * blkf
    w2p = jnp.concatenate(
        [w2_ref[...].T, jnp.zeros((H, 16 - O), jnp.float32)], axis=1)
    w28 = jnp.tile(w2p, (8, 8)) * blkf
    b1t = jnp.tile(b1_ref[0], (8,))
    b2t = jnp.tile(jnp.concatenate(
        [b2_ref[0], jnp.zeros((16 - O,), jnp.float32)]), (8,))

    cnt = cnt_ref[...].reshape(MLP_BLK // 128, 128)
    h = jnp.maximum(
        jnp.dot(cnt, m8, preferred_element_type=jnp.float32) + b1t[None, :],
        0.0)
    out = jnp.dot(h, w28, preferred_element_type=jnp.float32) + b2t[None, :]
    out_ref[...] = out


def _tc_mlp(counts_flat, table, W1, b1, W2, b2):
    out_pad = pl.pallas_call(
        _mlp_kernel,
        grid=(B // MLP_ROWS,),
        in_specs=[
            pl.BlockSpec((MLP_BLK,), lambda i: (i,)),
            pl.BlockSpec((VOCAB, E), lambda i: (0, 0)),
            pl.BlockSpec((H, E), lambda i: (0, 0)),
            pl.BlockSpec((1, H), lambda i: (0, 0)),
            pl.BlockSpec((O, H), lambda i: (0, 0)),
            pl.BlockSpec((1, O), lambda i: (0, 0)),
        ],
        out_specs=pl.BlockSpec((MLP_BLK // 128, 128), lambda i: (i, 0)),
        out_shape=jax.ShapeDtypeStruct((B * 16 // 128, 128), jnp.float32),
        compiler_params=pltpu.CompilerParams(
            dimension_semantics=("arbitrary",),
        ),
    )(counts_flat, table, W1, b1.reshape(1, H), W2, b2.reshape(1, O))
    return out_pad.reshape(B // 8, 8, 16)[:, :, :O].reshape(B, O)


def kernel(x, table, W1, b1, W2, b2):
    counts_flat = _sc_histogram(x.astype(jnp.int32))
    return _tc_mlp(counts_flat, table, W1, b1, W2, b2)


# speedup vs baseline: 1.1658x; 1.0293x over previous
import dataclasses

import jax
import jax.numpy as jnp
from jax import lax
from jax.experimental import pallas as pl
from jax.experimental.pallas import tpu as pltpu
from jax.experimental.pallas import tpu_sc as plsc

VOCAB = 13
L = 200
B = 16384
E = 32
H = 16
O = 4

NUM_TILES = 32
ROWS_PER_TILE = B // NUM_TILES
CHUNK_ROWS = 128
NUM_CHUNKS = ROWS_PER_TILE // CHUNK_ROWS
CHUNK_BINS = CHUNK_ROWS * 16
FULL_VECS = L // 16
TAIL_OFF = L - 16
TAIL_MASKED = 16 - (L - 16 * FULL_VECS)


def _sc_hist_kernel(x_hbm, cnt_hbm,
                    xv0, xv1, cnt0, cnt1,
                    s_in0, s_in1, s_out0, s_out1):
    wid = lax.axis_index("s") * 2 + lax.axis_index("c")
    row_base = wid * ROWS_PER_TILE

    ones = jnp.full((16,), 1.0, dtype=jnp.float32)
    zeros = jnp.zeros((16,), dtype=jnp.float32)
    tail_mask = jnp.arange(16, dtype=jnp.int32) >= TAIL_MASKED

    xbufs = [xv0, xv1]
    cbufs = [cnt0, cnt1]
    in_sems = [s_in0, s_in1]
    out_sems = [s_out0, s_out1]

    in_copies = [None, None]
    out_copies = [None, None]

    in_copies[0] = pltpu.async_copy(
        x_hbm.at[pl.ds(row_base, CHUNK_ROWS)], xbufs[0], in_sems[0])

    for ch in range(NUM_CHUNKS):
        p = ch % 2
        if ch + 1 < NUM_CHUNKS:
            in_copies[1 - p] = pltpu.async_copy(
                x_hbm.at[pl.ds(row_base + (ch + 1) * CHUNK_ROWS, CHUNK_ROWS)],
                xbufs[1 - p], in_sems[1 - p])
        if out_copies[p] is not None:
            out_copies[p].wait()
        in_copies[p].wait()
        xv = xbufs[p]
        cntv = cbufs[p]

        @plsc.parallel_loop(0, CHUNK_BINS, step=16, unroll=8)
        def _zero(j, cntv=cntv):
            cntv[pl.ds(j, 16)] = zeros

        @plsc.parallel_loop(0, CHUNK_ROWS, step=1, unroll=4)
        def _row(r, xv=xv, cntv=cntv):
            roff = jnp.full((16,), r * 16, dtype=jnp.int32)
            for k in range(FULL_VECS):
                vals = xv[r, pl.ds(k * 16, 16)]
                plsc.addupdate_scatter(cntv, [vals + roff], ones)
            tail = xv[r, pl.ds(TAIL_OFF, 16)]
            plsc.addupdate_scatter(cntv, [tail + roff], ones, mask=tail_mask)

        out_copies[p] = pltpu.async_copy(
            cntv,
            cnt_hbm.at[pl.ds((row_base + ch * CHUNK_ROWS) * 16, CHUNK_BINS)],
            out_sems[p])

    for p in range(2):
        if out_copies[p] is not None:
            out_copies[p].wait()


def _sc_histogram(x):
    mesh = plsc.VectorSubcoreMesh(core_axis_name="c", subcore_axis_name="s")
    cp = pltpu.CompilerParams()
    if "needs_layout_passes" in pltpu.CompilerParams.__dataclass_fields__:
        cp = dataclasses.replace(cp, needs_layout_passes=False)
    f = pl.kernel(
        _sc_hist_kernel,
        out_type=jax.ShapeDtypeStruct((B * 16,), jnp.float32),
        mesh=mesh,
        scratch_types=[
            pltpu.VMEM((CHUNK_ROWS, L), jnp.int32),
            pltpu.VMEM((CHUNK_ROWS, L), jnp.int32),
            pltpu.VMEM((CHUNK_BINS,), jnp.float32),
            pltpu.VMEM((CHUNK_BINS,), jnp.float32),
            pltpu.SemaphoreType.DMA,
            pltpu.SemaphoreType.DMA,
            pltpu.SemaphoreType.DMA,
            pltpu.SemaphoreType.DMA,
        ],
        compiler_params=cp,
    )
    return f(x)


MLP_ROWS = 4096
MLP_BLK = MLP_ROWS * 16


def _mlp_kernel(cnt_ref, table_ref, w1_ref, b1_ref, w2_ref, b2_ref, out_ref):
    m = jnp.dot(table_ref[...], w1_ref[...].T,
                preferred_element_type=jnp.float32) * (1.0 / L)
    mp = jnp.concatenate([m, jnp.zeros((16 - VOCAB, H), jnp.float32)], axis=0)

    blk = jnp.equal(lax.broadcasted_iota(jnp.int32, (128, 128), 0) // 16,
                    lax.broadcasted_iota(jnp.int32, (128, 128), 1) // 16)
    blkf = blk.astype(jnp.float32)
    m8 = jnp.tile(mp, (8, 8)) * blkf
    w2p = jnp.concatenate(
        [w2_ref[...].T, jnp.zeros((H, 16 - O), jnp.float32)], axis=1)
    w28 = jnp.tile(w2p, (8, 8)) * blkf
    b1t = jnp.tile(b1_ref[0], (8,))
    b2t = jnp.tile(jnp.concatenate(
        [b2_ref[0], jnp.zeros((16 - O,), jnp.float32)]), (8,))

    cnt = cnt_ref[...].reshape(MLP_BLK // 128, 128)
    h = jnp.maximum(
        jnp.dot(cnt, m8, preferred_element_type=jnp.float32) + b1t[None, :],
        0.0)
    out = jnp.dot(h, w28, preferred_element_type=jnp.float32) + b2t[None, :]
    out_ref[...] = out


def _tc_mlp(counts_flat, table, W1, b1, W2, b2):
    out_pad = pl.pallas_call(
        _mlp_kernel,
        grid=(B // MLP_ROWS,),
        in_specs=[
            pl.BlockSpec((MLP_BLK,), lambda i: (i,)),
            pl.BlockSpec((VOCAB, E), lambda i: (0, 0)),
            pl.BlockSpec((H, E), lambda i: (0, 0)),
            pl.BlockSpec((1, H), lambda i: (0, 0)),
            pl.BlockSpec((O, H), lambda i: (0, 0)),
            pl.BlockSpec((1, O), lambda i: (0, 0)),
        ],
        out_specs=pl.BlockSpec((MLP_BLK // 128, 128), lambda i: (i, 0)),
        out_shape=jax.ShapeDtypeStruct((B * 16 // 128, 128), jnp.float32),
        compiler_params=pltpu.CompilerParams(
            dimension_semantics=("arbitrary",),
        ),
    )(counts_flat, table, W1, b1.reshape(1, H), W2, b2.reshape(1, O))
    return out_pad.reshape(B // 8, 8, 16)[:, :, :O].reshape(B, O)


def kernel(x, table, W1, b1, W2, b2):
    counts_flat = _sc_histogram(x.astype(jnp.int32))
    return _tc_mlp(counts_flat, table, W1, b1, W2, b2)
